# baseline (speedup 1.0000x reference)
import functools

import jax
import jax.numpy as jnp
from jax import lax
from jax.experimental import pallas as pl
from jax.experimental.pallas import tpu as pltpu
from jax.experimental.pallas import tpu_sc as plsc

N = 10000
E = 320000
D = 128
HD = 64
EPS = 1e-5

NC = 2
NS = 16
ET = E // NS
C = 80
NCHUNK = ET // C
NPAD = 10240
RPT = NPAD // NS
ZR = 128
DGL = 16

_f32 = jnp.float32


NB = 5
PF = 4


def _sc_body(src_hbm, dst_hbm, y_hbm, out_p,
             srcb_v, dstb_v, rows_v, zrows_v, agg_sh, *sems):
    gsem = sems[:NB]
    ssem = sems[NB:2 * NB]

    cid = lax.axis_index("c")
    sid = lax.axis_index("s")

    pltpu.sync_copy(src_hbm.at[cid, sid], srcb_v)
    pltpu.sync_copy(dst_hbm.at[sid], dstb_v)

    zv = jnp.zeros((16,), _f32)

    def _zrow(i, _):
        for j in range(HD // 16):
            zrows_v[i, pl.ds(j * 16, 16)] = zv
        return 0
    lax.fori_loop(0, ZR, _zrow, 0)

    nbase = sid * RPT
    for j in range(RPT // ZR):
        pltpu.sync_copy(zrows_v, agg_sh.at[pl.ds(nbase + j * ZR, ZR)])

    plsc.subcore_barrier()

    for b in range(PF):
        pltpu.async_copy(y_hbm.at[srcb_v.at[b]], rows_v.at[b], gsem[b])

    def _outer(ko, _):
        for b in range(NB):
            k = ko * NB + b
            pltpu.make_async_copy(
                y_hbm.at[srcb_v.at[k]], rows_v.at[b], gsem[b]).wait()
            pltpu.make_async_copy(
                rows_v.at[b], agg_sh.at[dstb_v.at[k]], ssem[b]).start(add=True)
            bn = (b + PF) % NB
            kn = k + PF

            @pl.when(kn < NCHUNK)
            def _():
                @pl.when(kn - NB >= 0)
                def _():
                    pltpu.make_async_copy(
                        rows_v.at[bn], agg_sh.at[dstb_v.at[kn - NB]],
                        ssem[bn]).wait()
                pltpu.async_copy(y_hbm.at[srcb_v.at[kn]], rows_v.at[bn], gsem[bn])
        return 0
    lax.fori_loop(0, NCHUNK // NB, _outer, 0)

    for c in range(NCHUNK - NB, NCHUNK):
        b = c % NB
        pltpu.make_async_copy(
            rows_v.at[b], agg_sh.at[dstb_v.at[0]], ssem[b]).wait()

    plsc.subcore_barrier()

    pltpu.sync_copy(agg_sh.at[pl.ds(nbase, RPT)], out_p.at[cid, pl.ds(nbase, RPT)])


def _make_sc():
    mesh = plsc.VectorSubcoreMesh(core_axis_name="c", subcore_axis_name="s")
    scratch = [
        pltpu.VMEM((NCHUNK, C), jnp.int32),
        pltpu.VMEM((NCHUNK, C), jnp.int32),
        pltpu.VMEM((NB, C, HD), _f32),
        pltpu.VMEM((ZR, HD), _f32),
        pltpu.VMEM_SHARED((NPAD, HD), _f32),
    ] + [pltpu.SemaphoreType.DMA] * (2 * NB)
    return pl.kernel(
        _sc_body,
        out_type=jax.ShapeDtypeStruct((NC, NPAD, HD), _f32),
        mesh=mesh,
        scratch_types=scratch,
        compiler_params=pltpu.CompilerParams(use_tc_tiling_on_sc=False),
    )


NCD = NCHUNK // NC
NBD = 5


def _deg_body(dst_hbm, out_dg, dstb_v, ones_v, zdeg_v, deg_sh, *dsem):
    cid = lax.axis_index("c")
    sid = lax.axis_index("s")

    pltpu.sync_copy(dst_hbm.at[sid, pl.ds(cid * NCD, NCD)], dstb_v)

    zv = jnp.zeros((16,), _f32)

    def _zdeg(i, _):
        zdeg_v[i, pl.ds(0, 16)] = zv
        return 0
    lax.fori_loop(0, RPT, _zdeg, 0)
    nbase = sid * RPT
    pltpu.sync_copy(zdeg_v, deg_sh.at[pl.ds(nbase, RPT)])

    ov = jnp.ones((16,), _f32)

    def _onerow(i, _):
        ones_v[i, pl.ds(0, 16)] = ov
        return 0
    lax.fori_loop(0, C, _onerow, 0)

    plsc.subcore_barrier()

    def _outer(ko, _):
        for b in range(NBD):
            k = ko * NBD + b

            @pl.when(k - NBD >= 0)
            def _():
                pltpu.make_async_copy(
                    ones_v, deg_sh.at[dstb_v.at[k - NBD]], dsem[b]).wait()
            pltpu.make_async_copy(
                ones_v, deg_sh.at[dstb_v.at[k]], dsem[b]).start(add=True)
        return 0
    lax.fori_loop(0, NCD // NBD, _outer, 0)

    for c in range(NCD - NBD, NCD):
        pltpu.make_async_copy(
            ones_v, deg_sh.at[dstb_v.at[0]], dsem[c % NBD]).wait()

    plsc.subcore_barrier()

    pltpu.sync_copy(deg_sh.at[pl.ds(nbase, RPT)], out_dg.at[cid, pl.ds(nbase, RPT)])


def _make_deg():
    mesh = plsc.VectorSubcoreMesh(core_axis_name="c", subcore_axis_name="s")
    scratch = [
        pltpu.VMEM((NCD, C), jnp.int32),
        pltpu.VMEM((C, DGL), _f32),
        pltpu.VMEM((RPT, DGL), _f32),
        pltpu.VMEM_SHARED((NPAD, DGL), _f32),
    ] + [pltpu.SemaphoreType.DMA] * NBD
    return pl.kernel(
        _deg_body,
        out_type=jax.ShapeDtypeStruct((NC, NPAD, DGL), _f32),
        mesh=mesh,
        scratch_types=scratch,
        compiler_params=pltpu.CompilerParams(use_tc_tiling_on_sc=False),
    )


def _pre_body(x_ref, wl_ref, wr_ref, b_ref, y_ref, z_ref):
    c = pl.program_id(1)
    xb = x_ref[...]
    y_ref[...] = jnp.dot(xb, wl_ref[0], preferred_element_type=_f32)

    @pl.when(c == 0)
    def _():
        z_ref[...] = jnp.dot(xb, wr_ref[...], preferred_element_type=_f32) + b_ref[...]


def _mid_body(p_ref, dg_ref, z1_ref, gm_ref, bt_ref, rm_ref, rv_ref,
              wl_ref, wr_ref, b_ref, y_ref, z_ref):
    c = pl.program_id(1)
    deg = dg_ref[0, :, 0:1] + dg_ref[1, :, 0:1]
    dinv = 1.0 / jnp.maximum(deg, 1.0)
    agg = jnp.concatenate([p_ref[0], p_ref[1]], axis=1)
    hpre = agg * dinv + z1_ref[...]
    invstd = lax.rsqrt(rv_ref[...] + EPS)
    h = jnp.maximum((hpre - rm_ref[...]) * invstd * gm_ref[...] + bt_ref[...], 0.0)
    y_ref[...] = jnp.dot(h, wl_ref[0], preferred_element_type=_f32)

    @pl.when(c == 0)
    def _():
        z_ref[...] = jnp.dot(h, wr_ref[...], preferred_element_type=_f32) + b_ref[...]


def _fin_body(q_ref, dg_ref, z2_ref, wfc_ref, bfc_ref, out_ref):
    deg = dg_ref[0, :, 0:1] + dg_ref[1, :, 0:1]
    dinv = 1.0 / jnp.maximum(deg, 1.0)
    h2 = jnp.concatenate([q_ref[0], q_ref[1]], axis=1) * dinv + z2_ref[...]
    out_ref[...] = jnp.dot(h2, wfc_ref[...], preferred_element_type=_f32) + bfc_ref[...]


_RB = 2000
_GRID = N // _RB


def _row_spec2():
    return pl.BlockSpec((_RB, D), lambda i, c: (i, 0))


def _flat_plane_spec():
    return pl.BlockSpec((_RB, HD), lambda i, c: (c * _GRID + i, 0))


def _full_spec2(shape):
    return pl.BlockSpec(shape, lambda i, c: tuple(0 for _ in shape))


def _wl_spec():
    return pl.BlockSpec((1, D, HD), lambda i, c: (c, 0, 0))


_tc_pre = pl.pallas_call(
    _pre_body,
    grid=(_GRID, NC),
    in_specs=[_row_spec2(), _wl_spec(), _full_spec2((D, D)),
              _full_spec2((1, D))],
    out_specs=[_flat_plane_spec(), _row_spec2()],
    out_shape=[jax.ShapeDtypeStruct((NC * N, HD), _f32),
               jax.ShapeDtypeStruct((N, D), _f32)],
)

_tc_mid = pl.pallas_call(
    _mid_body,
    grid=(_GRID, NC),
    in_specs=[pl.BlockSpec((NC, _RB, HD), lambda i, c: (0, i, 0)),
              pl.BlockSpec((NC, _RB, DGL), lambda i, c: (0, i, 0)),
              _row_spec2()]
             + [_full_spec2((1, D))] * 4
             + [_wl_spec(), _full_spec2((D, D)), _full_spec2((1, D))],
    out_specs=[_flat_plane_spec(), _row_spec2()],
    out_shape=[jax.ShapeDtypeStruct((NC * N, HD), _f32),
               jax.ShapeDtypeStruct((N, D), _f32)],
)

_tc_fin = pl.pallas_call(
    _fin_body,
    grid=(_GRID,),
    in_specs=[pl.BlockSpec((NC, _RB, HD), lambda i: (0, i, 0)),
              pl.BlockSpec((NC, _RB, DGL), lambda i: (0, i, 0)),
              pl.BlockSpec((_RB, D), lambda i: (i, 0)),
              pl.BlockSpec((D, D), lambda i: (0, 0)),
              pl.BlockSpec((1, D), lambda i: (0, 0))],
    out_specs=pl.BlockSpec((_RB, D), lambda i: (i, 0)),
    out_shape=jax.ShapeDtypeStruct((N, D), _f32),
)

_sc_scatter = _make_sc()
_sc_deg = _make_deg()


def kernel(x, edge_index, Wl1, Wr1, b1, gamma, beta, rmean, rvar,
           Wl2, Wr2, b2, Wfc, bfc):
    src = edge_index[0].astype(jnp.int32)
    dst = edge_index[1].astype(jnp.int32)
    src2 = jnp.concatenate([src, src + N]).reshape(NC, NS, NCHUNK, C)
    dst3 = dst.reshape(NS, NCHUNK, C)
    r = lambda v: v.reshape(1, D)
    spl = lambda W: W.reshape(D, NC, HD).transpose(1, 0, 2)

    dg = _sc_deg(dst3)
    y1, z1 = _tc_pre(x, spl(Wl1), Wr1, r(b1))
    p = _sc_scatter(src2, dst3, y1)
    y2, z2 = _tc_mid(p, dg, z1, r(gamma), r(beta), r(rmean), r(rvar),
                     spl(Wl2), Wr2, r(b2))
    q = _sc_scatter(src2, dst3, y2)
    out = _tc_fin(q, dg, z2, Wfc, r(bfc))
    return out


# speedup vs baseline: 1.1334x; 1.1334x over previous
import jax
import jax.numpy as jnp
from jax import lax
from jax.experimental import pallas as pl
from jax.experimental.pallas import tpu as pltpu
from jax.experimental.pallas import tpu_sc as plsc

N = 10000
E = 320000
D = 128
HD = 64
EPS = 1e-5

NC = 2
NS = 16
ET = E // NS
C = 80
NCHUNK = ET // C
NPAD = 10240
RPT = NPAD // NS
ZR = 128
DGL = 16

_f32 = jnp.float32


NB = 5
PF = 4


def _sc_body(src_hbm, dst_hbm, y_hbm, out_p,
             srcb_v, dstb_v, rows_v, zrows_v, agg_sh, *sems):
    gsem = sems[:NB]
    ssem = sems[NB:2 * NB]

    cid = lax.axis_index("c")
    sid = lax.axis_index("s")

    yh = y_hbm

    pltpu.sync_copy(src_hbm.at[cid, sid], srcb_v)
    pltpu.sync_copy(dst_hbm.at[sid], dstb_v)

    zv = jnp.zeros((16,), _f32)

    def _zrow(i, _):
        for j in range(HD // 16):
            zrows_v[i, pl.ds(j * 16, 16)] = zv
        return 0
    lax.fori_loop(0, ZR, _zrow, 0)

    nbase = sid * RPT
    for j in range(RPT // ZR):
        pltpu.sync_copy(zrows_v, agg_sh.at[pl.ds(nbase + j * ZR, ZR)])

    plsc.subcore_barrier()

    for b in range(PF):
        pltpu.async_copy(yh.at[srcb_v.at[b]], rows_v.at[b], gsem[b])

    def _outer(ko, _):
        for b in range(NB):
            k = ko * NB + b
            pltpu.make_async_copy(
                yh.at[srcb_v.at[k]], rows_v.at[b], gsem[b]).wait()
            pltpu.make_async_copy(
                rows_v.at[b], agg_sh.at[dstb_v.at[k]], ssem[b]).start(add=True)
            bn = (b + PF) % NB
            kn = k + PF

            @pl.when(kn < NCHUNK)
            def _():
                @pl.when(kn - NB >= 0)
                def _():
                    pltpu.make_async_copy(
                        rows_v.at[bn], agg_sh.at[dstb_v.at[kn - NB]],
                        ssem[bn]).wait()
                pltpu.async_copy(yh.at[srcb_v.at[kn]], rows_v.at[bn], gsem[bn])
        return 0
    lax.fori_loop(0, NCHUNK // NB, _outer, 0)

    for c in range(NCHUNK - NB, NCHUNK):
        b = c % NB
        pltpu.make_async_copy(
            rows_v.at[b], agg_sh.at[dstb_v.at[0]], ssem[b]).wait()

    plsc.subcore_barrier()

    pltpu.sync_copy(agg_sh.at[pl.ds(nbase, RPT)],
                    out_p.at[pl.ds(nbase, RPT), pl.ds(cid * HD, HD)])


def _make_sc():
    mesh = plsc.VectorSubcoreMesh(core_axis_name="c", subcore_axis_name="s")
    scratch = [
        pltpu.VMEM((NCHUNK, C), jnp.int32),
        pltpu.VMEM((NCHUNK, C), jnp.int32),
        pltpu.VMEM((NB, C, HD), _f32),
        pltpu.VMEM((ZR, HD), _f32),
        pltpu.VMEM_SHARED((NPAD, HD), _f32),
    ] + [pltpu.SemaphoreType.DMA] * (2 * NB)
    return pl.kernel(
        _sc_body,
        out_type=jax.ShapeDtypeStruct((NPAD, D), _f32),
        mesh=mesh,
        scratch_types=scratch,
        compiler_params=pltpu.CompilerParams(use_tc_tiling_on_sc=False),
    )


NCD = NCHUNK // NC
NBD = 5


def _deg_body(dst_hbm, out_dg, dstb_v, ones_v, zdeg_v, deg_sh, *dsem):
    cid = lax.axis_index("c")
    sid = lax.axis_index("s")

    pltpu.sync_copy(dst_hbm.at[sid, pl.ds(cid * NCD, NCD)], dstb_v)

    zv = jnp.zeros((16,), _f32)

    def _zdeg(i, _):
        zdeg_v[i, pl.ds(0, 16)] = zv
        return 0
    lax.fori_loop(0, RPT, _zdeg, 0)
    nbase = sid * RPT
    pltpu.sync_copy(zdeg_v, deg_sh.at[pl.ds(nbase, RPT)])

    ov = jnp.ones((16,), _f32)

    def _onerow(i, _):
        ones_v[i, pl.ds(0, 16)] = ov
        return 0
    lax.fori_loop(0, C, _onerow, 0)

    plsc.subcore_barrier()

    def _outer(ko, _):
        for b in range(NBD):
            k = ko * NBD + b

            @pl.when(k - NBD >= 0)
            def _():
                pltpu.make_async_copy(
                    ones_v, deg_sh.at[dstb_v.at[k - NBD]], dsem[b]).wait()
            pltpu.make_async_copy(
                ones_v, deg_sh.at[dstb_v.at[k]], dsem[b]).start(add=True)
        return 0
    lax.fori_loop(0, NCD // NBD, _outer, 0)

    for c in range(NCD - NBD, NCD):
        pltpu.make_async_copy(
            ones_v, deg_sh.at[dstb_v.at[0]], dsem[c % NBD]).wait()

    plsc.subcore_barrier()

    pltpu.sync_copy(deg_sh.at[pl.ds(nbase, RPT)], out_dg.at[cid, pl.ds(nbase, RPT)])


def _make_deg():
    mesh = plsc.VectorSubcoreMesh(core_axis_name="c", subcore_axis_name="s")
    scratch = [
        pltpu.VMEM((NCD, C), jnp.int32),
        pltpu.VMEM((C, DGL), _f32),
        pltpu.VMEM((RPT, DGL), _f32),
        pltpu.VMEM_SHARED((NPAD, DGL), _f32),
    ] + [pltpu.SemaphoreType.DMA] * NBD
    return pl.kernel(
        _deg_body,
        out_type=jax.ShapeDtypeStruct((NC, NPAD, DGL), _f32),
        mesh=mesh,
        scratch_types=scratch,
        compiler_params=pltpu.CompilerParams(use_tc_tiling_on_sc=False),
    )


def _pre_body(x_ref, wls_ref, wr_ref, b_ref, y_ref, z_ref):
    xb = x_ref[...]
    y_ref[0] = jnp.dot(xb, wls_ref[0], preferred_element_type=_f32)
    y_ref[1] = jnp.dot(xb, wls_ref[1], preferred_element_type=_f32)
    z_ref[...] = jnp.dot(xb, wr_ref[...], preferred_element_type=_f32) + b_ref[...]


def _mid_body(p_ref, dg_ref, z1_ref, gm_ref, bt_ref, rm_ref, rv_ref,
              wls_ref, wr_ref, b_ref, y_ref, z_ref):
    deg = dg_ref[0, :, 0:1] + dg_ref[1, :, 0:1]
    dinv = 1.0 / jnp.maximum(deg, 1.0)
    hpre = p_ref[...] * dinv + z1_ref[...]
    invstd = lax.rsqrt(rv_ref[...] + EPS)
    h = jnp.maximum((hpre - rm_ref[...]) * invstd * gm_ref[...] + bt_ref[...], 0.0)
    y_ref[0] = jnp.dot(h, wls_ref[0], preferred_element_type=_f32)
    y_ref[1] = jnp.dot(h, wls_ref[1], preferred_element_type=_f32)
    z_ref[...] = jnp.dot(h, wr_ref[...], preferred_element_type=_f32) + b_ref[...]


def _fin_body(q_ref, dg_ref, z2_ref, wfc_ref, bfc_ref, out_ref):
    deg = dg_ref[0, :, 0:1] + dg_ref[1, :, 0:1]
    dinv = 1.0 / jnp.maximum(deg, 1.0)
    h2 = q_ref[...] * dinv + z2_ref[...]
    out_ref[...] = jnp.dot(h2, wfc_ref[...], preferred_element_type=_f32) + bfc_ref[...]


_RB = 2000
_GRID = N // _RB


def _row_spec():
    return pl.BlockSpec((_RB, D), lambda i: (i, 0))


def _deg_spec():
    return pl.BlockSpec((NC, _RB, DGL), lambda i: (0, i, 0))


def _full_spec(shape):
    return pl.BlockSpec(shape, lambda i: tuple(0 for _ in shape))


_tc_pre = pl.pallas_call(
    _pre_body,
    grid=(_GRID,),
    in_specs=[_row_spec(), _full_spec((NC, D, HD)), _full_spec((D, D)),
              _full_spec((1, D))],
    out_specs=[pl.BlockSpec((NC, _RB, HD), lambda i: (0, i, 0)), _row_spec()],
    out_shape=[jax.ShapeDtypeStruct((NC, N, HD), _f32),
               jax.ShapeDtypeStruct((N, D), _f32)],
)

_tc_mid = pl.pallas_call(
    _mid_body,
    grid=(_GRID,),
    in_specs=[_row_spec(), _deg_spec(), _row_spec()]
             + [_full_spec((1, D))] * 4
             + [_full_spec((NC, D, HD)), _full_spec((D, D)), _full_spec((1, D))],
    out_specs=[pl.BlockSpec((NC, _RB, HD), lambda i: (0, i, 0)), _row_spec()],
    out_shape=[jax.ShapeDtypeStruct((NC, N, HD), _f32),
               jax.ShapeDtypeStruct((N, D), _f32)],
)

_tc_fin = pl.pallas_call(
    _fin_body,
    grid=(_GRID,),
    in_specs=[_row_spec(), _deg_spec(), _row_spec(),
              _full_spec((D, D)), _full_spec((1, D))],
    out_specs=_row_spec(),
    out_shape=jax.ShapeDtypeStruct((N, D), _f32),
)

_sc_scatter = _make_sc()
_sc_deg = _make_deg()


def kernel(x, edge_index, Wl1, Wr1, b1, gamma, beta, rmean, rvar,
           Wl2, Wr2, b2, Wfc, bfc):
    src = edge_index[0].astype(jnp.int32)
    dst = edge_index[1].astype(jnp.int32)
    src2 = jnp.concatenate([src, src + N]).reshape(NC, NS, NCHUNK, C)
    dst3 = dst.reshape(NS, NCHUNK, C)
    r = lambda v: v.reshape(1, D)
    spl = lambda W: W.reshape(D, NC, HD).transpose(1, 0, 2)

    dg = _sc_deg(dst3)
    y1, z1 = _tc_pre(x, spl(Wl1), Wr1, r(b1))
    p = _sc_scatter(src2, dst3, y1.reshape(NC * N, HD))
    y2, z2 = _tc_mid(p, dg, z1, r(gamma), r(beta), r(rmean), r(rvar),
                     spl(Wl2), Wr2, r(b2))
    q = _sc_scatter(src2, dst3, y2.reshape(NC * N, HD))
    out = _tc_fin(q, dg, z2, Wfc, r(bfc))
    return out
